# baseline (speedup 1.0000x reference)
import functools

import jax
import jax.numpy as jnp
from jax import lax
from jax.experimental import pallas as pl
from jax.experimental.pallas import tpu as pltpu
from jax.experimental.pallas import tpu_sc as plsc

N = 10000
NNZ = 320000
E = 20000
H = 8
C = 16
D = 128
NEG = 0.2

_PREC = jax.lax.Precision.HIGHEST
CH = 128
N_CHUNKS = NNZ // CH
_F32 = jnp.float32


def _leaky(x):
    return jnp.where(x > 0, x, NEG * x)


def _wmax(gm_ref, sc2, i):
    bm = jnp.stack([jnp.broadcast_to(jnp.max(sc2[c], axis=0, keepdims=True),
                                     (8, 64)) for c in range(2)])

    @pl.when(i == 0)
    def _():
        gm_ref[...] = bm

    @pl.when(i > 0)
    def _():
        gm_ref[...] = jnp.maximum(gm_ref[...], bm)


def _tc_linear_body(x_ref, wc_ref, bt_ref, ac_ref, vt_ref, x0_ref, l_ref, gm_ref):
    i = pl.program_id(0)
    x = x_ref[...]
    vt = vt_ref[...]
    yall = jnp.dot(x, wc_ref[...], precision=_PREC)
    y = yall[:, 0:D] + bt_ref[0:1, :]
    for k in range(1, 4):
        yk = yall[:, D * k:D * (k + 1)] + bt_ref[k:k + 1, :]
        y = jnp.where(vt == k, yk, y)
    x0_ref[0] = y[:, 0:64]
    x0_ref[1] = y[:, 64:128]
    sc = _leaky(jnp.dot(y, ac_ref[...], precision=_PREC))
    sc2 = jnp.stack([sc[:, 0:64], sc[:, 64:128]])
    l_ref[...] = sc2
    _wmax(gm_ref, sc2, i)


def _tc_linear(X, Wc, bt, A_cat, vt2d):
    nb = 1000
    grid = N // nb
    return pl.pallas_call(
        _tc_linear_body,
        grid=(grid,),
        in_specs=[
            pl.BlockSpec((nb, D), lambda i: (i, 0)),
            pl.BlockSpec((D, 512), lambda i: (0, 0)),
            pl.BlockSpec((4, D), lambda i: (0, 0)),
            pl.BlockSpec((D, D), lambda i: (0, 0)),
            pl.BlockSpec((nb, 1), lambda i: (i, 0)),
        ],
        out_specs=[
            pl.BlockSpec((2, nb, 64), lambda i: (0, i, 0)),
            pl.BlockSpec((2, nb, 64), lambda i: (0, i, 0)),
            pl.BlockSpec((2, 8, 64), lambda i: (0, 0, 0)),
        ],
        out_shape=[
            jax.ShapeDtypeStruct((2, N, 64), _F32),
            jax.ShapeDtypeStruct((2, N, 64), _F32),
            jax.ShapeDtypeStruct((2, 8, 64), _F32),
        ],
    )(X, Wc, bt, A_cat, vt2d)


def _tc_exp_body(l_ref, gm_ref, out_ref):
    for c in range(2):
        g = gm_ref[c, 0:1, :]
        m = g[:, 0:16]
        for k in range(1, 4):
            m = jnp.maximum(m, g[:, 16 * k:16 * k + 16])
        gmb = jnp.concatenate([m, m, m, m], axis=1)
        out_ref[c] = jnp.exp(l_ref[c] - gmb)


def _tc_exp(L, gm):
    rows = L.shape[1]
    rb = 1000
    return pl.pallas_call(
        _tc_exp_body,
        grid=(rows // rb,),
        in_specs=[
            pl.BlockSpec((2, rb, 64), lambda i: (0, i, 0)),
            pl.BlockSpec((2, 8, 64), lambda i: (0, 0, 0)),
        ],
        out_specs=pl.BlockSpec((2, rb, 64), lambda i: (0, i, 0)),
        out_shape=jax.ShapeDtypeStruct((2, rows, 64), _F32),
    )(L, gm)


def _normalize_halves(a, relu):
    halves = []
    for c in range(2):
        cols = []
        for j in range(4):
            den = a[c, :, 71 - j:72 - j] + 1e-16
            v = a[c, :, 16 * j:16 * j + 16] / den
            cols.append(jnp.maximum(v, 0.0) if relu else v)
        halves.append(jnp.concatenate(cols, axis=1))
    return halves


def _tc_edge_body(acc_ref, av_ref, xe2_ref, l_ref, gm_ref):
    i = pl.program_id(0)
    halves = _normalize_halves(acc_ref[...], relu=True)
    xe2_ref[0] = halves[0]
    xe2_ref[1] = halves[1]
    xe_full = jnp.concatenate(halves, axis=1)
    sc = _leaky(jnp.dot(xe_full, av_ref[...], precision=_PREC))
    sc2 = jnp.stack([sc[:, 0:64], sc[:, 64:128]])
    l_ref[...] = sc2
    _wmax(gm_ref, sc2, i)


def _tc_edge(acc_e, A_cat):
    eb = 1000
    return pl.pallas_call(
        _tc_edge_body,
        grid=(E // eb,),
        in_specs=[
            pl.BlockSpec((2, eb, 72), lambda i: (0, i, 0)),
            pl.BlockSpec((D, D), lambda i: (0, 0)),
        ],
        out_specs=[
            pl.BlockSpec((2, eb, 64), lambda i: (0, i, 0)),
            pl.BlockSpec((2, eb, 64), lambda i: (0, i, 0)),
            pl.BlockSpec((2, 8, 64), lambda i: (0, 0, 0)),
        ],
        out_shape=[
            jax.ShapeDtypeStruct((2, E, 64), _F32),
            jax.ShapeDtypeStruct((2, E, 64), _F32),
            jax.ShapeDtypeStruct((2, 8, 64), _F32),
        ],
    )(acc_e, A_cat)


def _tc_final_body(acc_ref, out_ref):
    halves = _normalize_halves(acc_ref[...], relu=False)
    out_ref[:, 0:64] = halves[0]
    out_ref[:, 64:128] = halves[1]


def _tc_final(acc_v):
    nb = 1000
    return pl.pallas_call(
        _tc_final_body,
        grid=(N // nb,),
        in_specs=[pl.BlockSpec((2, nb, 72), lambda i: (0, i, 0))],
        out_specs=pl.BlockSpec((nb, D), lambda i: (i, 0)),
        out_shape=jax.ShapeDtypeStruct((N, D), _F32),
    )(acc_v)


def _sc_pass(table, expw, idxcat, zeros, nseg):
    mesh = plsc.VectorSubcoreMesh(core_axis_name="c", subcore_axis_name="s")
    q = ((nseg // 16) + 7) & ~7
    last = nseg - 15 * q
    assert last > 0 and last % 8 == 0

    @functools.partial(
        pl.kernel,
        mesh=mesh,
        out_type=jax.ShapeDtypeStruct((2, nseg, 72), _F32),
        scratch_types=[
            pltpu.VMEM_SHARED((nseg, 72), _F32),
            pltpu.VMEM((4, 3, CH), jnp.int32),
            pltpu.VMEM((2, CH, 16), _F32),
            pltpu.VMEM((2, CH, 64), _F32),
            pltpu.VMEM((2, CH, 72), _F32),
            pltpu.SemaphoreType.DMA,
            pltpu.SemaphoreType.DMA,
            pltpu.SemaphoreType.DMA,
            pltpu.SemaphoreType.DMA,
        ],
        compiler_params=pltpu.CompilerParams(use_tc_tiling_on_sc=False),
    )
    def k(table_h, expw_h, idxc_h, zeros_h, out_h,
          acc, ib, wb, xb, ob, si, sgw, sgx, ss):
        c = lax.axis_index("c")
        s = lax.axis_index("s")

        def stripe_copy(src, dst):
            off = pl.multiple_of(s * q, 8)

            @pl.when(s < 15)
            def _():
                pltpu.sync_copy(src.at[pl.ds(off, q)], dst.at[pl.ds(off, q)])

            @pl.when(s == 15)
            def _():
                pltpu.sync_copy(src.at[pl.ds(15 * q, last)],
                                dst.at[pl.ds(15 * q, last)])

        def issue_idx(i, slot):
            pltpu.async_copy(idxc_h.at[i * 16 + s], ib.at[slot], si)

        def wait_idx():
            pltpu.make_async_copy(idxc_h.at[0], ib.at[0], si).wait()

        def issue_gath(slot_i, slot_g):
            pltpu.async_copy(expw_h.at[c].at[ib.at[slot_i, 0]],
                             wb.at[slot_g], sgw)
            pltpu.async_copy(table_h.at[c].at[ib.at[slot_i, 2]],
                             xb.at[slot_g], sgx)

        def wait_gath():
            pltpu.make_async_copy(expw_h.at[0].at[ib.at[0, 0]],
                                  wb.at[0], sgw).wait()
            pltpu.make_async_copy(table_h.at[0].at[ib.at[0, 2]],
                                  xb.at[0], sgx).wait()

        def wait_scatter():
            pltpu.make_async_copy(ob.at[0], acc.at[ib.at[0, 1]], ss).wait()

        stripe_copy(zeros_h, acc)
        plsc.subcore_barrier()

        n_tile = (N_CHUNKS // 16
                  + jnp.where(s < N_CHUNKS % 16, 1, 0).astype(jnp.int32))

        issue_idx(0, 0)
        issue_idx(1, 1)
        wait_idx()
        issue_gath(0, 0)

        def chunk_body(i, carry):
            b = lax.rem(i, 2)
            b4 = lax.rem(i, 4)

            @pl.when(i >= 2)
            def _():
                wait_scatter()

            wait_gath()

            @pl.when(i + 1 < n_tile)
            def _():
                wait_idx()
                issue_gath(lax.rem(i + 1, 4), lax.rem(i + 1, 2))

            @pl.when(i + 2 < n_tile)
            def _():
                issue_idx(i + 2, lax.rem(i + 2, 4))

            for p in range(CH):
                wrow = wb[b, p, :]
                ob[b, p, pl.ds(56, 16)] = lax.rev(wrow, (0,))
                for h in range(4):
                    ob[b, p, pl.ds(16 * h, 16)] = (
                        xb[b, p, pl.ds(16 * h, 16)] * wrow[h])
            pltpu.async_copy(ob.at[b], acc.at[ib.at[b4, 1]], ss, add=True)
            return carry

        lax.fori_loop(0, n_tile, chunk_body, 0)
        wait_scatter()
        wait_scatter()
        plsc.subcore_barrier()
        stripe_copy(acc, out_h.at[c])

    return k(table, expw, idxcat, zeros)


def kernel(X, Wt, bt, att_e, att_v, vertex, edges, v_type,
           e_idx0, e_idx1, e_idx2, e_idx3, e_inv,
           v_idx0, v_idx1, v_idx2, v_idx3, v_inv):
    ce0 = e_idx0.shape[0]
    ce1 = ce0 + e_idx1.shape[0]
    ce2 = ce1 + e_idx2.shape[0]
    cv0 = v_idx0.shape[0]
    cv1 = cv0 + v_idx1.shape[0]
    cv2 = cv1 + v_idx2.shape[0]
    e_inv_i = e_inv.astype(jnp.int32)
    v_inv_i = v_inv.astype(jnp.int32)
    class_e = ((e_inv_i >= ce0).astype(jnp.int32)
               + (e_inv_i >= ce1).astype(jnp.int32)
               + (e_inv_i >= ce2).astype(jnp.int32))
    class_v = ((v_inv_i >= cv0).astype(jnp.int32)
               + (v_inv_i >= cv1).astype(jnp.int32)
               + (v_inv_i >= cv2).astype(jnp.int32))
    vertex32 = vertex.astype(jnp.int32)
    edges32 = edges.astype(jnp.int32)
    widx_e = vertex32 * 4 + class_e
    widx_v = edges32 * 4 + class_v

    eye8 = jnp.eye(8, dtype=_F32)

    def a_cat(att):
        parts = []
        for c in range(2):
            a = jnp.einsum('kjc,hj->hckj', att[:, c * 4:c * 4 + 4, :],
                           eye8[:, c * 4:c * 4 + 4])
            a = jnp.pad(a, ((0, 0), (0, 0), (0, 0), (0, 12)))
            parts.append(a.reshape(D, 64))
        return jnp.concatenate(parts, axis=1)

    A_e = a_cat(att_e)
    A_v = a_cat(att_v)
    Wc = Wt.transpose(1, 0, 2).reshape(D, 4 * D)

    def pack_idx(widx, seg, gat):
        return jnp.stack([widx.reshape(-1, CH), seg.reshape(-1, CH),
                          gat.reshape(-1, CH)], axis=1)

    idxcat_e = pack_idx(widx_e, edges32, vertex32)
    idxcat_v = pack_idx(widx_v, vertex32, edges32)

    vt2d = v_type.astype(jnp.int32).reshape(N, 1)
    xh2, L_e, gme = _tc_linear(X, Wc, bt, A_e, vt2d)
    expw = _tc_exp(L_e, gme).reshape(2, 4 * N, 16)

    acc_e = _sc_pass(xh2, expw, idxcat_e, jnp.zeros((E, 72), _F32), E)

    Xe2, L_v, gmv = _tc_edge(acc_e, A_v)
    expv = _tc_exp(L_v, gmv).reshape(2, 4 * E, 16)

    acc_v = _sc_pass(Xe2, expv, idxcat_v, jnp.zeros((N, 72), _F32), N)

    return _tc_final(acc_v)


# speedup vs baseline: 1.2792x; 1.2792x over previous
import functools

import jax
import jax.numpy as jnp
from jax import lax
from jax.experimental import pallas as pl
from jax.experimental.pallas import tpu as pltpu
from jax.experimental.pallas import tpu_sc as plsc

N = 10000
NNZ = 320000
E = 20000
H = 8
C = 16
D = 128
NEG = 0.2

_PREC = jax.lax.Precision.HIGHEST
CH_E = 80
CH_V = 128
_F32 = jnp.float32


def _leaky(x):
    return jnp.where(x > 0, x, NEG * x)


def _wmax(gm_ref, sc2, i):
    bm = jnp.stack([jnp.broadcast_to(jnp.max(sc2[c], axis=0, keepdims=True),
                                     (8, 64)) for c in range(2)])

    @pl.when(i == 0)
    def _():
        gm_ref[...] = bm

    @pl.when(i > 0)
    def _():
        gm_ref[...] = jnp.maximum(gm_ref[...], bm)


def _tc_linear_body(x_ref, wc_ref, bt_ref, ac_ref, vt_ref, x0_ref, l_ref, gm_ref):
    i = pl.program_id(0)
    x = x_ref[...]
    vt = vt_ref[...]
    yall = jnp.dot(x, wc_ref[...], precision=_PREC)
    y = yall[:, 0:D] + bt_ref[0:1, :]
    for k in range(1, 4):
        yk = yall[:, D * k:D * (k + 1)] + bt_ref[k:k + 1, :]
        y = jnp.where(vt == k, yk, y)
    x0_ref[0] = y[:, 0:64]
    x0_ref[1] = y[:, 64:128]
    sc = _leaky(jnp.dot(y, ac_ref[...], precision=_PREC))
    sc2 = jnp.stack([sc[:, 0:64], sc[:, 64:128]])
    l_ref[...] = sc2
    _wmax(gm_ref, sc2, i)


def _tc_linear(X, Wc, bt, A_cat, vt2d):
    nb = 1000
    grid = N // nb
    return pl.pallas_call(
        _tc_linear_body,
        grid=(grid,),
        in_specs=[
            pl.BlockSpec((nb, D), lambda i: (i, 0)),
            pl.BlockSpec((D, 512), lambda i: (0, 0)),
            pl.BlockSpec((4, D), lambda i: (0, 0)),
            pl.BlockSpec((D, D), lambda i: (0, 0)),
            pl.BlockSpec((nb, 1), lambda i: (i, 0)),
        ],
        out_specs=[
            pl.BlockSpec((2, nb, 64), lambda i: (0, i, 0)),
            pl.BlockSpec((2, nb, 64), lambda i: (0, i, 0)),
            pl.BlockSpec((2, 8, 64), lambda i: (0, 0, 0)),
        ],
        out_shape=[
            jax.ShapeDtypeStruct((2, N, 64), _F32),
            jax.ShapeDtypeStruct((2, N, 64), _F32),
            jax.ShapeDtypeStruct((2, 8, 64), _F32),
        ],
    )(X, Wc, bt, A_cat, vt2d)


def _tc_exp_body(l_ref, gm_ref, out_ref):
    for c in range(2):
        g = gm_ref[c, 0:1, :]
        m = g[:, 0:16]
        for k in range(1, 4):
            m = jnp.maximum(m, g[:, 16 * k:16 * k + 16])
        gmb = jnp.concatenate([m, m, m, m], axis=1)
        out_ref[c] = jnp.exp(l_ref[c] - gmb)


def _tc_exp(L, gm):
    rows = L.shape[1]
    rb = 1000
    return pl.pallas_call(
        _tc_exp_body,
        grid=(rows // rb,),
        in_specs=[
            pl.BlockSpec((2, rb, 64), lambda i: (0, i, 0)),
            pl.BlockSpec((2, 8, 64), lambda i: (0, 0, 0)),
        ],
        out_specs=pl.BlockSpec((2, rb, 64), lambda i: (0, i, 0)),
        out_shape=jax.ShapeDtypeStruct((2, rows, 64), _F32),
    )(L, gm)


def _normalize_halves(a, relu):
    halves = []
    for c in range(2):
        cols = []
        for j in range(4):
            den = a[c, :, 64 + j:65 + j] + 1e-16
            v = a[c, :, 16 * j:16 * j + 16] / den
            cols.append(jnp.maximum(v, 0.0) if relu else v)
        halves.append(jnp.concatenate(cols, axis=1))
    return halves


def _tc_edge_body(acc_ref, av_ref, xe2_ref, l_ref, gm_ref):
    i = pl.program_id(0)
    halves = _normalize_halves(acc_ref[...], relu=True)
    xe2_ref[0] = halves[0]
    xe2_ref[1] = halves[1]
    xe_full = jnp.concatenate(halves, axis=1)
    sc = _leaky(jnp.dot(xe_full, av_ref[...], precision=_PREC))
    sc2 = jnp.stack([sc[:, 0:64], sc[:, 64:128]])
    l_ref[...] = sc2
    _wmax(gm_ref, sc2, i)


def _tc_edge(acc_e, A_cat):
    eb = 1000
    return pl.pallas_call(
        _tc_edge_body,
        grid=(E // eb,),
        in_specs=[
            pl.BlockSpec((2, eb, 80), lambda i: (0, i, 0)),
            pl.BlockSpec((D, D), lambda i: (0, 0)),
        ],
        out_specs=[
            pl.BlockSpec((2, eb, 64), lambda i: (0, i, 0)),
            pl.BlockSpec((2, eb, 64), lambda i: (0, i, 0)),
            pl.BlockSpec((2, 8, 64), lambda i: (0, 0, 0)),
        ],
        out_shape=[
            jax.ShapeDtypeStruct((2, E, 64), _F32),
            jax.ShapeDtypeStruct((2, E, 64), _F32),
            jax.ShapeDtypeStruct((2, 8, 64), _F32),
        ],
    )(acc_e, A_cat)


def _tc_final_body(acc_ref, out_ref):
    halves = _normalize_halves(acc_ref[...], relu=False)
    out_ref[:, 0:64] = halves[0]
    out_ref[:, 64:128] = halves[1]


def _tc_final(acc_v):
    nb = 1000
    return pl.pallas_call(
        _tc_final_body,
        grid=(N // nb,),
        in_specs=[pl.BlockSpec((2, nb, 80), lambda i: (0, i, 0))],
        out_specs=pl.BlockSpec((nb, D), lambda i: (i, 0)),
        out_shape=jax.ShapeDtypeStruct((N, D), _F32),
    )(acc_v)


def _sc_pass(table, expw, idxcat, zeros, nseg, ch):
    mesh = plsc.VectorSubcoreMesh(core_axis_name="c", subcore_axis_name="s")
    q = ((nseg // 16) + 7) & ~7
    last = nseg - 15 * q
    assert last > 0 and last % 8 == 0

    @functools.partial(
        pl.kernel,
        mesh=mesh,
        out_type=jax.ShapeDtypeStruct((2, nseg, 80), _F32),
        scratch_types=[
            pltpu.VMEM_SHARED((nseg, 80), _F32),
            pltpu.VMEM((4, 3, ch), jnp.int32),
            pltpu.VMEM((2, ch, 16), _F32),
            pltpu.VMEM((2, ch, 64), _F32),
            pltpu.VMEM((2, ch, 80), _F32),
            pltpu.SemaphoreType.DMA,
            pltpu.SemaphoreType.DMA,
            pltpu.SemaphoreType.DMA,
            pltpu.SemaphoreType.DMA,
        ],
        compiler_params=pltpu.CompilerParams(use_tc_tiling_on_sc=False),
    )
    def k(table_h, expw_h, idxc_h, zeros_h, out_h,
          acc, ib, wb, xb, ob, si, sgw, sgx, ss):
        c = lax.axis_index("c")
        s = lax.axis_index("s")

        def stripe_copy(src, dst):
            off = pl.multiple_of(s * q, 8)

            @pl.when(s < 15)
            def _():
                pltpu.sync_copy(src.at[pl.ds(off, q)], dst.at[pl.ds(off, q)])

            @pl.when(s == 15)
            def _():
                pltpu.sync_copy(src.at[pl.ds(15 * q, last)],
                                dst.at[pl.ds(15 * q, last)])

        def issue_idx(i, slot):
            pltpu.async_copy(idxc_h.at[i * 16 + s], ib.at[slot], si)

        def wait_idx():
            pltpu.make_async_copy(idxc_h.at[0], ib.at[0], si).wait()

        def issue_gath(slot_i, slot_g):
            pltpu.async_copy(expw_h.at[c].at[ib.at[slot_i, 0]],
                             wb.at[slot_g], sgw)
            pltpu.async_copy(table_h.at[c].at[ib.at[slot_i, 2]],
                             xb.at[slot_g], sgx)

        def wait_gath():
            pltpu.make_async_copy(expw_h.at[0].at[ib.at[0, 0]],
                                  wb.at[0], sgw).wait()
            pltpu.make_async_copy(table_h.at[0].at[ib.at[0, 2]],
                                  xb.at[0], sgx).wait()

        def wait_scatter():
            pltpu.make_async_copy(ob.at[0], acc.at[ib.at[0, 1]], ss).wait()

        stripe_copy(zeros_h, acc)
        plsc.subcore_barrier()

        n_chunks = NNZ // ch
        n_tile = (n_chunks // 16
                  + jnp.where(s < n_chunks % 16, 1, 0).astype(jnp.int32))

        issue_idx(0, 0)
        issue_idx(1, 1)
        wait_idx()
        issue_gath(0, 0)

        def chunk_body(i, carry):
            b = lax.rem(i, 2)
            b4 = lax.rem(i, 4)

            @pl.when(i >= 2)
            def _():
                wait_scatter()

            wait_gath()

            @pl.when(i + 1 < n_tile)
            def _():
                wait_idx()
                issue_gath(lax.rem(i + 1, 4), lax.rem(i + 1, 2))

            @pl.when(i + 2 < n_tile)
            def _():
                issue_idx(i + 2, lax.rem(i + 2, 4))

            for p in range(ch):
                wrow = wb[b, p, :]
                for h in range(4):
                    ob[b, p, pl.ds(16 * h, 16)] = (
                        xb[b, p, pl.ds(16 * h, 16)] * wrow[h])
                ob[b, p, pl.ds(64, 16)] = wrow
            pltpu.async_copy(ob.at[b], acc.at[ib.at[b4, 1]], ss, add=True)
            return carry

        lax.fori_loop(0, n_tile, chunk_body, 0)
        wait_scatter()
        wait_scatter()
        plsc.subcore_barrier()
        stripe_copy(acc, out_h.at[c])

    return k(table, expw, idxcat, zeros)


def kernel(X, Wt, bt, att_e, att_v, vertex, edges, v_type,
           e_idx0, e_idx1, e_idx2, e_idx3, e_inv,
           v_idx0, v_idx1, v_idx2, v_idx3, v_inv):
    ce0 = e_idx0.shape[0]
    ce1 = ce0 + e_idx1.shape[0]
    ce2 = ce1 + e_idx2.shape[0]
    cv0 = v_idx0.shape[0]
    cv1 = cv0 + v_idx1.shape[0]
    cv2 = cv1 + v_idx2.shape[0]
    e_inv_i = e_inv.astype(jnp.int32)
    v_inv_i = v_inv.astype(jnp.int32)
    class_e = ((e_inv_i >= ce0).astype(jnp.int32)
               + (e_inv_i >= ce1).astype(jnp.int32)
               + (e_inv_i >= ce2).astype(jnp.int32))
    class_v = ((v_inv_i >= cv0).astype(jnp.int32)
               + (v_inv_i >= cv1).astype(jnp.int32)
               + (v_inv_i >= cv2).astype(jnp.int32))
    vertex32 = vertex.astype(jnp.int32)
    edges32 = edges.astype(jnp.int32)
    widx_e = vertex32 * 4 + class_e
    widx_v = edges32 * 4 + class_v

    eye8 = jnp.eye(8, dtype=_F32)

    def a_cat(att):
        parts = []
        for c in range(2):
            a = jnp.einsum('kjc,hj->hckj', att[:, c * 4:c * 4 + 4, :],
                           eye8[:, c * 4:c * 4 + 4])
            a = jnp.pad(a, ((0, 0), (0, 0), (0, 0), (0, 12)))
            parts.append(a.reshape(D, 64))
        return jnp.concatenate(parts, axis=1)

    A_e = a_cat(att_e)
    A_v = a_cat(att_v)
    Wc = Wt.transpose(1, 0, 2).reshape(D, 4 * D)

    def pack_idx(widx, seg, gat, ch):
        return jnp.stack([widx.reshape(-1, ch), seg.reshape(-1, ch),
                          gat.reshape(-1, ch)], axis=1)

    idxcat_e = pack_idx(widx_e, edges32, vertex32, CH_E)
    idxcat_v = pack_idx(widx_v, vertex32, edges32, CH_V)

    vt2d = v_type.astype(jnp.int32).reshape(N, 1)
    xh2, L_e, gme = _tc_linear(X, Wc, bt, A_e, vt2d)
    expw = _tc_exp(L_e, gme).reshape(2, 4 * N, 16)

    acc_e = _sc_pass(xh2, expw, idxcat_e, jnp.zeros((E, 80), _F32), E, CH_E)

    Xe2, L_v, gmv = _tc_edge(acc_e, A_v)
    expv = _tc_exp(L_v, gmv).reshape(2, 4 * E, 16)

    acc_v = _sc_pass(Xe2, expv, idxcat_v, jnp.zeros((N, 80), _F32), N, CH_V)

    return _tc_final(acc_v)


# speedup vs baseline: 1.5440x; 1.2071x over previous
import functools

import jax
import jax.numpy as jnp
from jax import lax
from jax.experimental import pallas as pl
from jax.experimental.pallas import tpu as pltpu
from jax.experimental.pallas import tpu_sc as plsc

N = 10000
NNZ = 320000
E = 20000
H = 8
C = 16
D = 128
NEG = 0.2

_PREC = jax.lax.Precision.DEFAULT
CH = 80
N_ITER = (NNZ // CH + 15) // 16
NNZ_P = N_ITER * 16 * CH
_F32 = jnp.float32


def _leaky(x):
    return jnp.where(x > 0, x, NEG * x)


def _wmax(gm_ref, sc2, i):
    bm = jnp.stack([jnp.broadcast_to(jnp.max(sc2[c], axis=0, keepdims=True),
                                     (8, 64)) for c in range(2)])

    @pl.when(i == 0)
    def _():
        gm_ref[...] = bm

    @pl.when(i > 0)
    def _():
        gm_ref[...] = jnp.maximum(gm_ref[...], bm)


def _tc_linear_body(x_ref, wc_ref, bt_ref, ac_ref, vt_ref, x0_ref, l_ref, gm_ref):
    i = pl.program_id(0)
    x = x_ref[...]
    vt = vt_ref[...]
    yall = jnp.dot(x, wc_ref[...], precision=_PREC)
    y = yall[:, 0:D] + bt_ref[0:1, :]
    for k in range(1, 4):
        yk = yall[:, D * k:D * (k + 1)] + bt_ref[k:k + 1, :]
        y = jnp.where(vt == k, yk, y)
    x0_ref[0] = y[:, 0:64]
    x0_ref[1] = y[:, 64:128]
    sc = _leaky(jnp.dot(y, ac_ref[...], precision=_PREC))
    sc2 = jnp.stack([sc[:, 0:64], sc[:, 64:128]])
    l_ref[...] = sc2
    _wmax(gm_ref, sc2, i)


def _tc_linear(X, Wc, bt, A_cat, vt2d):
    nb = 1000
    grid = N // nb
    return pl.pallas_call(
        _tc_linear_body,
        grid=(grid,),
        in_specs=[
            pl.BlockSpec((nb, D), lambda i: (i, 0)),
            pl.BlockSpec((D, 512), lambda i: (0, 0)),
            pl.BlockSpec((4, D), lambda i: (0, 0)),
            pl.BlockSpec((D, D), lambda i: (0, 0)),
            pl.BlockSpec((nb, 1), lambda i: (i, 0)),
        ],
        out_specs=[
            pl.BlockSpec((2, nb, 64), lambda i: (0, i, 0)),
            pl.BlockSpec((2, nb, 64), lambda i: (0, i, 0)),
            pl.BlockSpec((2, 8, 64), lambda i: (0, 0, 0)),
        ],
        out_shape=[
            jax.ShapeDtypeStruct((2, N, 64), _F32),
            jax.ShapeDtypeStruct((2, N, 64), _F32),
            jax.ShapeDtypeStruct((2, 8, 64), _F32),
        ],
    )(X, Wc, bt, A_cat, vt2d)


def _tc_exp_body(l_ref, gm_ref, out_ref):
    for c in range(2):
        g = gm_ref[c, 0:1, :]
        m = g[:, 0:16]
        for k in range(1, 4):
            m = jnp.maximum(m, g[:, 16 * k:16 * k + 16])
        gmb = jnp.concatenate([m, m, m, m], axis=1)
        out_ref[c] = jnp.exp(l_ref[c] - gmb)


def _tc_exp(L, gm):
    rows = L.shape[1]
    rb = 1000
    return pl.pallas_call(
        _tc_exp_body,
        grid=(rows // rb,),
        in_specs=[
            pl.BlockSpec((2, rb, 64), lambda i: (0, i, 0)),
            pl.BlockSpec((2, 8, 64), lambda i: (0, 0, 0)),
        ],
        out_specs=pl.BlockSpec((2, rb, 64), lambda i: (0, i, 0)),
        out_shape=jax.ShapeDtypeStruct((2, rows, 64), _F32),
    )(L, gm)


def _normalize_halves(a, relu):
    halves = []
    for c in range(2):
        cols = []
        for j in range(4):
            den = a[c, :, 64 + j:65 + j] + 1e-16
            v = a[c, :, 16 * j:16 * j + 16] / den
            cols.append(jnp.maximum(v, 0.0) if relu else v)
        halves.append(jnp.concatenate(cols, axis=1))
    return halves


def _tc_edge_body(acc_ref, av_ref, xe2_ref, l_ref, gm_ref):
    i = pl.program_id(0)
    halves = _normalize_halves(acc_ref[...], relu=True)
    xe2_ref[0] = halves[0]
    xe2_ref[1] = halves[1]
    xe_full = jnp.concatenate(halves, axis=1)
    sc = _leaky(jnp.dot(xe_full, av_ref[...], precision=_PREC))
    sc2 = jnp.stack([sc[:, 0:64], sc[:, 64:128]])
    l_ref[...] = sc2
    _wmax(gm_ref, sc2, i)


def _tc_edge(acc_e, A_cat):
    eb = 1000
    return pl.pallas_call(
        _tc_edge_body,
        grid=(E // eb,),
        in_specs=[
            pl.BlockSpec((2, eb, 80), lambda i: (0, i, 0)),
            pl.BlockSpec((D, D), lambda i: (0, 0)),
        ],
        out_specs=[
            pl.BlockSpec((2, eb, 64), lambda i: (0, i, 0)),
            pl.BlockSpec((2, eb, 64), lambda i: (0, i, 0)),
            pl.BlockSpec((2, 8, 64), lambda i: (0, 0, 0)),
        ],
        out_shape=[
            jax.ShapeDtypeStruct((2, E, 64), _F32),
            jax.ShapeDtypeStruct((2, E, 64), _F32),
            jax.ShapeDtypeStruct((2, 8, 64), _F32),
        ],
    )(acc_e, A_cat)


def _tc_final_body(acc_ref, out_ref):
    halves = _normalize_halves(acc_ref[...], relu=False)
    out_ref[:, 0:64] = halves[0]
    out_ref[:, 64:128] = halves[1]


def _tc_final(acc_v):
    nb = 1000
    return pl.pallas_call(
        _tc_final_body,
        grid=(N // nb,),
        in_specs=[pl.BlockSpec((2, nb, 80), lambda i: (0, i, 0))],
        out_specs=pl.BlockSpec((nb, D), lambda i: (i, 0)),
        out_shape=jax.ShapeDtypeStruct((N, D), _F32),
    )(acc_v)


def _sc_pass(table, expw, idxcat, zeros, nseg, n_iter):
    mesh = plsc.VectorSubcoreMesh(core_axis_name="c", subcore_axis_name="s")
    q = ((nseg // 16) + 7) & ~7
    last = nseg - 15 * q
    assert last > 0 and last % 8 == 0

    @functools.partial(
        pl.kernel,
        mesh=mesh,
        out_type=jax.ShapeDtypeStruct((2, nseg, 80), _F32),
        scratch_types=[
            pltpu.VMEM_SHARED((nseg, 80), _F32),
            pltpu.VMEM((4, 3, CH), jnp.int32),
            pltpu.VMEM((2, CH, 16), _F32),
            pltpu.VMEM((2, CH, 64), _F32),
            pltpu.VMEM((2, CH, 80), _F32),
            pltpu.SemaphoreType.DMA,
            pltpu.SemaphoreType.DMA,
            pltpu.SemaphoreType.DMA,
            pltpu.SemaphoreType.DMA,
        ],
        compiler_params=pltpu.CompilerParams(use_tc_tiling_on_sc=False),
    )
    def k(table_h, expw_h, idxc_h, zeros_h, out_h,
          acc, ib, wb, xb, ob, si, sgw, sgx, ss):
        c = lax.axis_index("c")
        s = lax.axis_index("s")

        def stripe_copy(src, dst):
            off = pl.multiple_of(s * q, 8)

            @pl.when(s < 15)
            def _():
                pltpu.sync_copy(src.at[pl.ds(off, q)], dst.at[pl.ds(off, q)])

            @pl.when(s == 15)
            def _():
                pltpu.sync_copy(src.at[pl.ds(15 * q, last)],
                                dst.at[pl.ds(15 * q, last)])

        def issue_idx(i, slot):
            pltpu.async_copy(idxc_h.at[i * 16 + s], ib.at[slot], si)

        def wait_idx():
            pltpu.make_async_copy(idxc_h.at[0], ib.at[0], si).wait()

        def issue_gath(slot_i, slot_g):
            pltpu.async_copy(expw_h.at[c].at[ib.at[slot_i, 0]],
                             wb.at[slot_g], sgw)
            pltpu.async_copy(table_h.at[c].at[ib.at[slot_i, 2]],
                             xb.at[slot_g], sgx)

        def wait_gath():
            pltpu.make_async_copy(expw_h.at[0].at[ib.at[0, 0]],
                                  wb.at[0], sgw).wait()
            pltpu.make_async_copy(table_h.at[0].at[ib.at[0, 2]],
                                  xb.at[0], sgx).wait()

        def wait_scatter():
            pltpu.make_async_copy(ob.at[0], acc.at[ib.at[0, 1]], ss).wait()

        stripe_copy(zeros_h, acc)
        plsc.subcore_barrier()

        issue_idx(0, 0)
        issue_idx(1, 1)
        wait_idx()
        issue_gath(0, 0)

        def chunk_body(i, carry):
            b = lax.rem(i, 2)
            b4 = lax.rem(i, 4)

            @pl.when(i >= 2)
            def _():
                wait_scatter()

            wait_gath()

            @pl.when(i + 1 < n_iter)
            def _():
                wait_idx()
                issue_gath(lax.rem(i + 1, 4), lax.rem(i + 1, 2))

            @pl.when(i + 2 < n_iter)
            def _():
                issue_idx(i + 2, lax.rem(i + 2, 4))

            for p in range(CH):
                wrow = wb[b, p, :]
                for h in range(4):
                    ob[b, p, pl.ds(16 * h, 16)] = (
                        xb[b, p, pl.ds(16 * h, 16)] * wrow[h])
                ob[b, p, pl.ds(64, 16)] = wrow
            pltpu.async_copy(ob.at[b], acc.at[ib.at[b4, 1]], ss, add=True)
            return carry

        lax.fori_loop(0, n_iter, chunk_body, 0)
        wait_scatter()
        wait_scatter()
        plsc.subcore_barrier()
        stripe_copy(acc, out_h.at[c])

    return k(table, expw, idxcat, zeros)


def kernel(X, Wt, bt, att_e, att_v, vertex, edges, v_type,
           e_idx0, e_idx1, e_idx2, e_idx3, e_inv,
           v_idx0, v_idx1, v_idx2, v_idx3, v_inv):
    ce0 = e_idx0.shape[0]
    ce1 = ce0 + e_idx1.shape[0]
    ce2 = ce1 + e_idx2.shape[0]
    cv0 = v_idx0.shape[0]
    cv1 = cv0 + v_idx1.shape[0]
    cv2 = cv1 + v_idx2.shape[0]
    e_inv_i = e_inv.astype(jnp.int32)
    v_inv_i = v_inv.astype(jnp.int32)
    class_e = ((e_inv_i >= ce0).astype(jnp.int32)
               + (e_inv_i >= ce1).astype(jnp.int32)
               + (e_inv_i >= ce2).astype(jnp.int32))
    class_v = ((v_inv_i >= cv0).astype(jnp.int32)
               + (v_inv_i >= cv1).astype(jnp.int32)
               + (v_inv_i >= cv2).astype(jnp.int32))
    vertex32 = vertex.astype(jnp.int32)
    edges32 = edges.astype(jnp.int32)
    widx_e = vertex32 * 4 + class_e
    widx_v = edges32 * 4 + class_v

    eye8 = jnp.eye(8, dtype=_F32)

    def a_cat(att):
        parts = []
        for c in range(2):
            a = jnp.einsum('kjc,hj->hckj', att[:, c * 4:c * 4 + 4, :],
                           eye8[:, c * 4:c * 4 + 4])
            a = jnp.pad(a, ((0, 0), (0, 0), (0, 0), (0, 12)))
            parts.append(a.reshape(D, 64))
        return jnp.concatenate(parts, axis=1)

    A_e = a_cat(att_e)
    A_v = a_cat(att_v)
    Wc = Wt.transpose(1, 0, 2).reshape(D, 4 * D)

    pad = NNZ_P - NNZ
    j8 = jnp.arange(pad, dtype=jnp.int32) % 8

    def pack_idx(widx, seg, gat, zrow_base):
        if pad:
            widx = jnp.concatenate([widx, zrow_base + j8])
            seg = jnp.concatenate([seg, j8])
            gat = jnp.concatenate([gat, j8])
        return jnp.stack([widx.reshape(-1, CH), seg.reshape(-1, CH),
                          gat.reshape(-1, CH)], axis=1)

    idxcat_e = pack_idx(widx_e, edges32, vertex32, 4 * N)
    idxcat_v = pack_idx(widx_v, vertex32, edges32, 4 * E)

    vt2d = v_type.astype(jnp.int32).reshape(N, 1)
    xh2, L_e, gme = _tc_linear(X, Wc, bt, A_e, vt2d)
    expw = _tc_exp(L_e, gme).reshape(2, 4 * N, 16)

    acc_e = _sc_pass(xh2, expw, idxcat_e, jnp.zeros((E, 80), _F32), E, N_ITER)

    Xe2, L_v, gmv = _tc_edge(acc_e, A_v)
    expv = _tc_exp(L_v, gmv).reshape(2, 4 * E, 16)

    acc_v = _sc_pass(Xe2, expv, idxcat_v, jnp.zeros((N, 80), _F32), N, N_ITER)

    return _tc_final(acc_v)


# speedup vs baseline: 1.5721x; 1.0182x over previous
import functools

import jax
import jax.numpy as jnp
from jax import lax
from jax.experimental import pallas as pl
from jax.experimental.pallas import tpu as pltpu
from jax.experimental.pallas import tpu_sc as plsc

N = 10000
NNZ = 320000
E = 20000
H = 8
C = 16
D = 128
NEG = 0.2

_PREC = jax.lax.Precision.DEFAULT
CH = 80
N_ITER = (NNZ // CH + 15) // 16
NNZ_P = N_ITER * 16 * CH
_F32 = jnp.float32


def _leaky(x):
    return jnp.where(x > 0, x, NEG * x)


def _wmax(gm_ref, sc2, i):
    bm = jnp.stack([jnp.broadcast_to(jnp.max(sc2[c], axis=0, keepdims=True),
                                     (8, 64)) for c in range(2)])

    @pl.when(i == 0)
    def _():
        gm_ref[...] = bm

    @pl.when(i > 0)
    def _():
        gm_ref[...] = jnp.maximum(gm_ref[...], bm)


def _tc_linear_body(x_ref, wc_ref, bt_ref, ac_ref, vt_ref, x0_ref, l_ref, gm_ref):
    i = pl.program_id(0)
    x = x_ref[...]
    vt = vt_ref[...]
    yall = jnp.dot(x, wc_ref[...], precision=_PREC)
    y = yall[:, 0:D] + bt_ref[0:1, :]
    for k in range(1, 4):
        yk = yall[:, D * k:D * (k + 1)] + bt_ref[k:k + 1, :]
        y = jnp.where(vt == k, yk, y)
    x0_ref[0] = y[:, 0:64]
    x0_ref[1] = y[:, 64:128]
    sc = _leaky(jnp.dot(y, ac_ref[...], precision=_PREC))
    sc2 = jnp.stack([sc[:, 0:64], sc[:, 64:128]])
    l_ref[...] = sc2
    _wmax(gm_ref, sc2, i)


def _tc_linear(X, Wc, bt, A_cat, vt2d):
    nb = 2000
    grid = N // nb
    return pl.pallas_call(
        _tc_linear_body,
        grid=(grid,),
        in_specs=[
            pl.BlockSpec((nb, D), lambda i: (i, 0)),
            pl.BlockSpec((D, 512), lambda i: (0, 0)),
            pl.BlockSpec((4, D), lambda i: (0, 0)),
            pl.BlockSpec((D, D), lambda i: (0, 0)),
            pl.BlockSpec((nb, 1), lambda i: (i, 0)),
        ],
        out_specs=[
            pl.BlockSpec((2, nb, 64), lambda i: (0, i, 0)),
            pl.BlockSpec((2, nb, 64), lambda i: (0, i, 0)),
            pl.BlockSpec((2, 8, 64), lambda i: (0, 0, 0)),
        ],
        out_shape=[
            jax.ShapeDtypeStruct((2, N, 64), _F32),
            jax.ShapeDtypeStruct((2, N, 64), _F32),
            jax.ShapeDtypeStruct((2, 8, 64), _F32),
        ],
    )(X, Wc, bt, A_cat, vt2d)


def _tc_exp_body(l_ref, gm_ref, out_ref):
    for c in range(2):
        g = gm_ref[c, 0:1, :]
        m = g[:, 0:16]
        for k in range(1, 4):
            m = jnp.maximum(m, g[:, 16 * k:16 * k + 16])
        gmb = jnp.concatenate([m, m, m, m], axis=1)
        out_ref[c] = jnp.exp(l_ref[c] - gmb)


def _tc_exp(L, gm):
    rows = L.shape[1]
    rb = 2000
    return pl.pallas_call(
        _tc_exp_body,
        grid=(rows // rb,),
        in_specs=[
            pl.BlockSpec((2, rb, 64), lambda i: (0, i, 0)),
            pl.BlockSpec((2, 8, 64), lambda i: (0, 0, 0)),
        ],
        out_specs=pl.BlockSpec((2, rb, 64), lambda i: (0, i, 0)),
        out_shape=jax.ShapeDtypeStruct((2, rows, 64), _F32),
    )(L, gm)


def _normalize_halves(a, relu):
    halves = []
    for c in range(2):
        cols = []
        for j in range(4):
            den = a[c, :, 64 + j:65 + j] + 1e-16
            v = a[c, :, 16 * j:16 * j + 16] / den
            cols.append(jnp.maximum(v, 0.0) if relu else v)
        halves.append(jnp.concatenate(cols, axis=1))
    return halves


def _tc_edge_body(acc_ref, av_ref, xe2_ref, l_ref, gm_ref):
    i = pl.program_id(0)
    halves = _normalize_halves(acc_ref[...], relu=True)
    xe2_ref[0] = halves[0]
    xe2_ref[1] = halves[1]
    xe_full = jnp.concatenate(halves, axis=1)
    sc = _leaky(jnp.dot(xe_full, av_ref[...], precision=_PREC))
    sc2 = jnp.stack([sc[:, 0:64], sc[:, 64:128]])
    l_ref[...] = sc2
    _wmax(gm_ref, sc2, i)


def _tc_edge(acc_e, A_cat):
    eb = 2000
    return pl.pallas_call(
        _tc_edge_body,
        grid=(E // eb,),
        in_specs=[
            pl.BlockSpec((2, eb, 80), lambda i: (0, i, 0)),
            pl.BlockSpec((D, D), lambda i: (0, 0)),
        ],
        out_specs=[
            pl.BlockSpec((2, eb, 64), lambda i: (0, i, 0)),
            pl.BlockSpec((2, eb, 64), lambda i: (0, i, 0)),
            pl.BlockSpec((2, 8, 64), lambda i: (0, 0, 0)),
        ],
        out_shape=[
            jax.ShapeDtypeStruct((2, E, 64), _F32),
            jax.ShapeDtypeStruct((2, E, 64), _F32),
            jax.ShapeDtypeStruct((2, 8, 64), _F32),
        ],
    )(acc_e, A_cat)


def _tc_final_body(acc_ref, out_ref):
    halves = _normalize_halves(acc_ref[...], relu=False)
    out_ref[:, 0:64] = halves[0]
    out_ref[:, 64:128] = halves[1]


def _tc_final(acc_v):
    nb = 2000
    return pl.pallas_call(
        _tc_final_body,
        grid=(N // nb,),
        in_specs=[pl.BlockSpec((2, nb, 80), lambda i: (0, i, 0))],
        out_specs=pl.BlockSpec((nb, D), lambda i: (i, 0)),
        out_shape=jax.ShapeDtypeStruct((N, D), _F32),
    )(acc_v)


def _sc_pass(table, expw, idxcat, zeros, nseg, n_iter):
    mesh = plsc.VectorSubcoreMesh(core_axis_name="c", subcore_axis_name="s")
    q = ((nseg // 16) + 7) & ~7
    last = nseg - 15 * q
    assert last > 0 and last % 8 == 0

    @functools.partial(
        pl.kernel,
        mesh=mesh,
        out_type=jax.ShapeDtypeStruct((2, nseg, 80), _F32),
        scratch_types=[
            pltpu.VMEM_SHARED((nseg, 80), _F32),
            pltpu.VMEM((4, 3, CH), jnp.int32),
            pltpu.VMEM((2, CH, 16), _F32),
            pltpu.VMEM((2, CH, 64), _F32),
            pltpu.VMEM((2, CH, 80), _F32),
            pltpu.SemaphoreType.DMA,
            pltpu.SemaphoreType.DMA,
            pltpu.SemaphoreType.DMA,
            pltpu.SemaphoreType.DMA,
        ],
        compiler_params=pltpu.CompilerParams(use_tc_tiling_on_sc=False),
    )
    def k(table_h, expw_h, idxc_h, zeros_h, out_h,
          acc, ib, wb, xb, ob, si, sgw, sgx, ss):
        c = lax.axis_index("c")
        s = lax.axis_index("s")

        def stripe_copy(src, dst):
            off = pl.multiple_of(s * q, 8)

            @pl.when(s < 15)
            def _():
                pltpu.sync_copy(src.at[pl.ds(off, q)], dst.at[pl.ds(off, q)])

            @pl.when(s == 15)
            def _():
                pltpu.sync_copy(src.at[pl.ds(15 * q, last)],
                                dst.at[pl.ds(15 * q, last)])

        def issue_idx(i, slot):
            pltpu.async_copy(idxc_h.at[i * 16 + s], ib.at[slot], si)

        def wait_idx():
            pltpu.make_async_copy(idxc_h.at[0], ib.at[0], si).wait()

        def issue_gath(slot_i, slot_g):
            pltpu.async_copy(expw_h.at[c].at[ib.at[slot_i, 0]],
                             wb.at[slot_g], sgw)
            pltpu.async_copy(table_h.at[c].at[ib.at[slot_i, 2]],
                             xb.at[slot_g], sgx)

        def wait_gath():
            pltpu.make_async_copy(expw_h.at[0].at[ib.at[0, 0]],
                                  wb.at[0], sgw).wait()
            pltpu.make_async_copy(table_h.at[0].at[ib.at[0, 2]],
                                  xb.at[0], sgx).wait()

        def wait_scatter():
            pltpu.make_async_copy(ob.at[0], acc.at[ib.at[0, 1]], ss).wait()

        stripe_copy(zeros_h, acc)
        plsc.subcore_barrier()

        issue_idx(0, 0)
        issue_idx(1, 1)
        wait_idx()
        issue_gath(0, 0)

        def chunk_body(i, carry):
            b = lax.rem(i, 2)
            b4 = lax.rem(i, 4)

            @pl.when(i >= 2)
            def _():
                wait_scatter()

            wait_gath()

            @pl.when(i + 1 < n_iter)
            def _():
                wait_idx()
                issue_gath(lax.rem(i + 1, 4), lax.rem(i + 1, 2))

            @pl.when(i + 2 < n_iter)
            def _():
                issue_idx(i + 2, lax.rem(i + 2, 4))

            for p in range(CH):
                wrow = wb[b, p, :]
                for h in range(4):
                    ob[b, p, pl.ds(16 * h, 16)] = (
                        xb[b, p, pl.ds(16 * h, 16)] * wrow[h])
                ob[b, p, pl.ds(64, 16)] = wrow
            pltpu.async_copy(ob.at[b], acc.at[ib.at[b4, 1]], ss, add=True)
            return carry

        lax.fori_loop(0, n_iter, chunk_body, 0)
        wait_scatter()
        wait_scatter()
        plsc.subcore_barrier()
        stripe_copy(acc, out_h.at[c])

    return k(table, expw, idxcat, zeros)


def kernel(X, Wt, bt, att_e, att_v, vertex, edges, v_type,
           e_idx0, e_idx1, e_idx2, e_idx3, e_inv,
           v_idx0, v_idx1, v_idx2, v_idx3, v_inv):
    ce0 = e_idx0.shape[0]
    ce1 = ce0 + e_idx1.shape[0]
    ce2 = ce1 + e_idx2.shape[0]
    cv0 = v_idx0.shape[0]
    cv1 = cv0 + v_idx1.shape[0]
    cv2 = cv1 + v_idx2.shape[0]
    e_inv_i = e_inv.astype(jnp.int32)
    v_inv_i = v_inv.astype(jnp.int32)
    class_e = ((e_inv_i >= ce0).astype(jnp.int32)
               + (e_inv_i >= ce1).astype(jnp.int32)
               + (e_inv_i >= ce2).astype(jnp.int32))
    class_v = ((v_inv_i >= cv0).astype(jnp.int32)
               + (v_inv_i >= cv1).astype(jnp.int32)
               + (v_inv_i >= cv2).astype(jnp.int32))
    vertex32 = vertex.astype(jnp.int32)
    edges32 = edges.astype(jnp.int32)
    widx_e = vertex32 * 4 + class_e
    widx_v = edges32 * 4 + class_v

    eye8 = jnp.eye(8, dtype=_F32)

    def a_cat(att):
        parts = []
        for c in range(2):
            a = jnp.einsum('kjc,hj->hckj', att[:, c * 4:c * 4 + 4, :],
                           eye8[:, c * 4:c * 4 + 4])
            a = jnp.pad(a, ((0, 0), (0, 0), (0, 0), (0, 12)))
            parts.append(a.reshape(D, 64))
        return jnp.concatenate(parts, axis=1)

    A_e = a_cat(att_e)
    A_v = a_cat(att_v)
    Wc = Wt.transpose(1, 0, 2).reshape(D, 4 * D)

    pad = NNZ_P - NNZ
    j8 = jnp.arange(pad, dtype=jnp.int32) % 8

    def pack_idx(widx, seg, gat, zrow_base):
        if pad:
            widx = jnp.concatenate([widx, zrow_base + j8])
            seg = jnp.concatenate([seg, j8])
            gat = jnp.concatenate([gat, j8])
        return jnp.stack([widx.reshape(-1, CH), seg.reshape(-1, CH),
                          gat.reshape(-1, CH)], axis=1)

    idxcat_e = pack_idx(widx_e, edges32, vertex32, 4 * N)
    idxcat_v = pack_idx(widx_v, vertex32, edges32, 4 * E)

    vt2d = v_type.astype(jnp.int32).reshape(N, 1)
    xh2, L_e, gme = _tc_linear(X, Wc, bt, A_e, vt2d)
    expw = _tc_exp(L_e, gme).reshape(2, 4 * N, 16)

    acc_e = _sc_pass(xh2, expw, idxcat_e, jnp.zeros((E, 80), _F32), E, N_ITER)

    Xe2, L_v, gmv = _tc_edge(acc_e, A_v)
    expv = _tc_exp(L_v, gmv).reshape(2, 4 * E, 16)

    acc_v = _sc_pass(Xe2, expv, idxcat_v, jnp.zeros((N, 80), _F32), N, N_ITER)

    return _tc_final(acc_v)
